# baseline (speedup 1.0000x reference)
import functools

import jax
import jax.numpy as jnp
from jax import lax
from jax.experimental import pallas as pl
from jax.experimental.pallas import tpu as pltpu
from jax.experimental.pallas import tpu_sc as plsc

_N_ROWS, _N_COLS = 8192, 4096
_LANES = 16
_NUM_WORKERS = 32
_GROUPS = _N_ROWS // _NUM_WORKERS // _LANES
_CC = 1024
_CHUNKS = _N_COLS // _CC
_TILES = _GROUPS * _CHUNKS


@functools.cache
def _sc_cumsum_call():
    mesh = plsc.VectorSubcoreMesh(core_axis_name="c", subcore_axis_name="s")

    @functools.partial(
        pl.kernel,
        mesh=mesh,
        compiler_params=pltpu.CompilerParams(needs_layout_passes=False),
        out_type=jax.ShapeDtypeStruct((_N_ROWS, _N_COLS), jnp.float32),
        scratch_types=[pltpu.VMEM((_LANES, _CC), jnp.float32),
                       pltpu.VMEM((_LANES, _CC), jnp.float32),
                       pltpu.VMEM((_LANES, _CC), jnp.float32),
                       pltpu.VMEM((_LANES, _CC), jnp.float32),
                       pltpu.SemaphoreType.DMA,
                       pltpu.SemaphoreType.DMA,
                       pltpu.SemaphoreType.DMA,
                       pltpu.SemaphoreType.DMA],
    )
    def sc_cumsum(x_hbm, o_hbm, ibuf0, ibuf1, obuf0, obuf1,
                  sem_i0, sem_i1, sem_o0, sem_o1):
        wid = lax.axis_index("s") * 2 + lax.axis_index("c")
        ibufs, obufs = (ibuf0, ibuf1), (obuf0, obuf1)
        sems_i, sems_o = (sem_i0, sem_i1), (sem_o0, sem_o1)

        def tile_src(t):
            g = t // _CHUNKS
            c = lax.rem(t, _CHUNKS)
            row0 = (wid * _GROUPS + g) * _LANES
            return x_hbm.at[pl.ds(row0, _LANES), pl.ds(c * _CC, _CC)]

        def tile_dst(t):
            g = t // _CHUNKS
            c = lax.rem(t, _CHUNKS)
            row0 = (wid * _GROUPS + g) * _LANES
            return o_hbm.at[pl.ds(row0, _LANES), pl.ds(c * _CC, _CC)]

        pltpu.async_copy(tile_src(0), ibuf0, sem_i0)

        last = lax.iota(jnp.int32, _LANES) * 0 + (_LANES - 1)

        def step(i, carries):
            for b in range(2):
                t = 2 * i + b
                ib, ob = ibufs[b], obufs[b]

                @pl.when((t + 1 < _TILES) if b == 0 else (i < _TILES // 2 - 1))
                def _():
                    pltpu.async_copy(tile_src(t + 1), ibufs[1 - b],
                                     sems_i[1 - b])

                pltpu.make_async_copy(tile_src(t), ib, sems_i[b]).wait()

                @pl.when(i >= 1)
                def _():
                    pltpu.make_async_copy(ob, tile_dst(t), sems_o[b]).wait()

                maskf = (lax.rem(t, _CHUNKS) != 0).astype(jnp.float32)
                carries = tuple(cr * maskf for cr in carries)

                def col_body(j, carries):
                    new = []
                    for r in range(_LANES):
                        v = ib[r, pl.ds(j * _LANES, _LANES)]
                        s = plsc.cumsum(v) + carries[r]
                        ob[r, pl.ds(j * _LANES, _LANES)] = s
                        new.append(s.at[last].get(mode="promise_in_bounds"))
                    return tuple(new)

                carries = plsc.parallel_loop(
                    0, _CC // _LANES, unroll=4, carry=carries)(col_body)

                pltpu.async_copy(ob, tile_dst(t), sems_o[b])
            return carries

        lax.fori_loop(0, _TILES // 2, step,
                      (jnp.zeros((_LANES,), jnp.float32),) * _LANES)

        pltpu.make_async_copy(obuf0, tile_dst(_TILES - 2), sem_o0).wait()
        pltpu.make_async_copy(obuf1, tile_dst(_TILES - 1), sem_o1).wait()

    return sc_cumsum


def kernel(x):
    return _sc_cumsum_call()(x)
